# baseline (speedup 1.0000x reference)
import jax
import jax.numpy as jnp
from jax.experimental import pallas as pl
from jax.experimental.pallas import tpu as pltpu
from jax.experimental.pallas import tpu_sc as plsc

VOCAB = 100000
D = 64
B = 1024
CTX = 10

VT = 2048
NV = -(-VOCAB // VT)
NB = 2
BB = B // NB

_GATHER_WIN = 128


def _sc_gather(emb2, x_flat):
    n = x_flat.shape[1]
    mesh = plsc.VectorSubcoreMesh(core_axis_name="c", subcore_axis_name="s")

    @pl.kernel(
        out_type=jax.ShapeDtypeStruct((n, 2 * D), emb2.dtype),
        mesh=mesh,
        scratch_types=[pltpu.VMEM((1, _GATHER_WIN), jnp.int32)],
    )
    def gather_kernel(emb_hbm, i_hbm, o_hbm, tmp_ref):
        def body(i_vmem, o_vmem):
            @pl.loop(0, _GATHER_WIN, step=16)
            def _(c):
                sl = (0, pl.ds(c, 16))
                tmp_ref[sl] = jax.lax.shift_right_logical(i_vmem[sl], 1)

            pltpu.sync_copy(emb_hbm.at[tmp_ref.at[0]], o_vmem)

        pltpu.emit_pipeline(
            body,
            grid=(n // _GATHER_WIN,),
            in_specs=[pl.BlockSpec((1, _GATHER_WIN), index_map=lambda i: (0, i))],
            out_specs=[pl.BlockSpec((_GATHER_WIN, 2 * D), index_map=lambda i: (i, 0))],
            core_axis_name=("c", "s"),
            dimension_semantics=(pltpu.PARALLEL,),
        )(i_hbm, o_hbm)

    return gather_kernel(emb2, x_flat)


def _tc_body(g_ref, x_ref, w_ref, b_ref, o_ref, s_ref, m_ref, l_ref):
    p = pl.program_id(1)
    v = pl.program_id(2)

    @pl.when((p == 0) & (v == 0))
    def _():
        acc = jnp.zeros((BB, D), jnp.float32)
        for j in range(CTX):
            left = g_ref[:, j * 2 * D:j * 2 * D + D]
            right = g_ref[:, j * 2 * D + D:(j + 1) * 2 * D]
            odd = (x_ref[:, j:j + 1] & 1) == 1
            acc = acc + jnp.where(odd, right, left)
        s_ref[...] = acc.astype(jnp.bfloat16)
        m_ref[...] = jnp.full((BB, 1), -jnp.inf, jnp.float32)
        l_ref[...] = jnp.zeros((BB, 1), jnp.float32)

    logits = jax.lax.dot_general(
        s_ref[...], w_ref[...],
        (((1,), (1,)), ((), ())),
        preferred_element_type=jnp.float32,
    ) + b_ref[...]

    @pl.when(p == 0)
    def _():
        col = jax.lax.broadcasted_iota(jnp.int32, (1, VT), 1) + v * VT
        lm = jnp.where(col < VOCAB, logits, -jnp.inf)
        m_prev = m_ref[...]
        m_new = jnp.maximum(m_prev, jnp.max(lm, axis=1, keepdims=True))
        l_ref[...] = (l_ref[...] * jnp.exp(m_prev - m_new)
                      + jnp.sum(jnp.exp(lm - m_new), axis=1, keepdims=True))
        m_ref[...] = m_new

    @pl.when(p == 1)
    def _():
        o_ref[...] = logits - (m_ref[...] + jnp.log(l_ref[...]))


def _tc_call(g2, x, w16, b2):
    return pl.pallas_call(
        _tc_body,
        grid=(NB, 2, NV),
        in_specs=[
            pl.BlockSpec((BB, CTX * 2 * D), lambda nb, p, v: (nb, 0)),
            pl.BlockSpec((BB, CTX), lambda nb, p, v: (nb, 0)),
            pl.BlockSpec((VT, D), lambda nb, p, v: (v, 0)),
            pl.BlockSpec((1, VT), lambda nb, p, v: (0, v)),
        ],
        out_specs=pl.BlockSpec((BB, VT), lambda nb, p, v: (nb, v * p)),
        out_shape=jax.ShapeDtypeStruct((B, VOCAB), jnp.float32),
        scratch_shapes=[
            pltpu.VMEM((BB, D), jnp.bfloat16),
            pltpu.VMEM((BB, 1), jnp.float32),
            pltpu.VMEM((BB, 1), jnp.float32),
        ],
        compiler_params=pltpu.CompilerParams(
            dimension_semantics=("parallel", "arbitrary", "arbitrary"),
        ),
    )(g2, x, w16, b2)


def kernel(x, emb, W, b):
    x = x.astype(jnp.int32)
    x_flat = x.reshape(1, B * CTX)
    emb2 = emb.reshape(VOCAB // 2, 2 * D)
    g = _sc_gather(emb2, x_flat)
    g2 = g.reshape(B, CTX * 2 * D)
    w16 = W.astype(jnp.bfloat16)
    b2 = b.reshape(1, VOCAB)
    return _tc_call(g2, x, w16, b2)


# speedup vs baseline: 1.0149x; 1.0149x over previous
import jax
import jax.numpy as jnp
from jax.experimental import pallas as pl
from jax.experimental.pallas import tpu as pltpu
from jax.experimental.pallas import tpu_sc as plsc

VOCAB = 100000
D = 64
B = 1024
CTX = 10

VT = 2048
NV = -(-VOCAB // VT)
VPAD = NV * VT
NB = 2
BB = B // NB

LOG2E = 1.4426950408889634

_GATHER_WIN = 128
_STAT_CHUNK = 8192


def _sc_gather(emb2, x_flat):
    n = x_flat.shape[1]
    mesh = plsc.VectorSubcoreMesh(core_axis_name="c", subcore_axis_name="s")

    @pl.kernel(
        out_type=jax.ShapeDtypeStruct((n, 2 * D), emb2.dtype),
        mesh=mesh,
        scratch_types=[pltpu.VMEM((1, _GATHER_WIN), jnp.int32)],
    )
    def gather_kernel(emb_hbm, i_hbm, o_hbm, tmp_ref):
        def body(i_vmem, o_vmem):
            @pl.loop(0, _GATHER_WIN, step=16)
            def _(c):
                sl = (0, pl.ds(c, 16))
                tmp_ref[sl] = jax.lax.shift_right_logical(i_vmem[sl], 1)

            pltpu.sync_copy(emb_hbm.at[tmp_ref.at[0]], o_vmem)

        pltpu.emit_pipeline(
            body,
            grid=(n // _GATHER_WIN,),
            in_specs=[pl.BlockSpec((1, _GATHER_WIN), index_map=lambda i: (0, i))],
            out_specs=[pl.BlockSpec((_GATHER_WIN, 2 * D), index_map=lambda i: (i, 0))],
            core_axis_name=("c", "s"),
            dimension_semantics=(pltpu.PARALLEL,),
        )(i_hbm, o_hbm)

    return gather_kernel(emb2, x_flat)


def _stats_body(w_ref, o_ref):
    m = jnp.float32(0.0)
    for k in range(VPAD // _STAT_CHUNK):
        c = w_ref[k * _STAT_CHUNK:(k + 1) * _STAT_CHUNK, 0:D].astype(jnp.float32)
        m = jnp.maximum(m, jnp.max(jnp.sum(c * c, axis=1)))
    mb = jnp.max(w_ref[:, D:D + 1].astype(jnp.float32))
    lane = jax.lax.broadcasted_iota(jnp.int32, (1, 128), 1)
    o_ref[...] = jnp.where(lane == 0, jnp.sqrt(m),
                           jnp.where(lane == 1, mb, 0.0))


def _stats_call(w_aug):
    return pl.pallas_call(
        _stats_body,
        out_shape=jax.ShapeDtypeStruct((1, 128), jnp.float32),
    )(w_aug)


def _tc_body(stats_ref, g_ref, x_ref, wt_ref, o_ref,
             s_ref, mhat_ref, mhat2_ref, lse_ref, acc_ref):
    p = pl.program_id(1)
    v = pl.program_id(2)

    @pl.when((p == 0) & (v == 0))
    def _():
        acc = jnp.zeros((BB, D), jnp.float32)
        for j in range(CTX):
            left = g_ref[:, j * 2 * D:j * 2 * D + D].astype(jnp.float32)
            right = g_ref[:, j * 2 * D + D:(j + 1) * 2 * D].astype(jnp.float32)
            odd = (x_ref[:, j:j + 1] & 1) == 1
            acc = acc + jnp.where(odd, right, left)
        s_ref[:, 0:D] = acc.astype(jnp.bfloat16)
        s_ref[:, D:D + 1] = jnp.ones((BB, 1), jnp.bfloat16)
        snorm = jnp.sqrt(jnp.sum(acc * acc, axis=1, keepdims=True))
        mhat = snorm * stats_ref[0, 0] + stats_ref[0, 1]
        mhat_ref[...] = mhat
        mhat2_ref[...] = mhat * LOG2E
        acc_ref[...] = jnp.zeros((BB, 128), jnp.float32)

    logits = jax.lax.dot_general(
        s_ref[...], wt_ref[...],
        (((1,), (0,)), ((), ())),
        preferred_element_type=jnp.float32,
    )

    @pl.when(p == 0)
    def _():
        e2 = jnp.exp2(logits * LOG2E - mhat2_ref[...])
        part = acc_ref[...]
        for k in range(VT // 128):
            part = part + e2[:, k * 128:(k + 1) * 128]
        acc_ref[...] = part

    @pl.when((p == 1) & (v == 0))
    def _():
        lse_ref[...] = mhat_ref[...] + jnp.log(
            jnp.sum(acc_ref[...], axis=1, keepdims=True))

    @pl.when(p == 1)
    def _():
        o_ref[...] = logits - lse_ref[...]


def _tc_call(stats, g2, x, wt):
    return pl.pallas_call(
        _tc_body,
        grid=(NB, 2, NV),
        in_specs=[
            pl.BlockSpec((1, 128), lambda nb, p, v: (0, 0)),
            pl.BlockSpec((BB, CTX * 2 * D), lambda nb, p, v: (nb, 0)),
            pl.BlockSpec((BB, CTX), lambda nb, p, v: (nb, 0)),
            pl.BlockSpec((D + 1, VT), lambda nb, p, v: (0, v)),
        ],
        out_specs=pl.BlockSpec((BB, VT), lambda nb, p, v: (nb, v * p)),
        out_shape=jax.ShapeDtypeStruct((B, VOCAB), jnp.float32),
        scratch_shapes=[
            pltpu.VMEM((BB, D + 1), jnp.bfloat16),
            pltpu.VMEM((BB, 1), jnp.float32),
            pltpu.VMEM((BB, 1), jnp.float32),
            pltpu.VMEM((BB, 1), jnp.float32),
            pltpu.VMEM((BB, 128), jnp.float32),
        ],
        compiler_params=pltpu.CompilerParams(
            dimension_semantics=("parallel", "arbitrary", "arbitrary"),
        ),
    )(stats, g2, x, wt)


def kernel(x, emb, W, b):
    x = x.astype(jnp.int32)
    x_flat = x.reshape(1, B * CTX)
    emb2 = emb.reshape(VOCAB // 2, 2 * D)
    g = _sc_gather(emb2, x_flat)
    g2 = g.reshape(B, CTX * 2 * D)

    w_aug = jnp.concatenate([W, b[:, None]], axis=1).astype(jnp.bfloat16)
    w_aug = jnp.pad(w_aug, ((0, VPAD - VOCAB), (0, 0)))
    w_aug = w_aug.at[VOCAB:, D].set(jnp.bfloat16(-1e9))
    stats = _stats_call(w_aug)
    wt = w_aug.T
    return _tc_call(stats, g2, x, wt)


# speedup vs baseline: 1.1448x; 1.1279x over previous
import jax
import jax.numpy as jnp
from jax.experimental import pallas as pl
from jax.experimental.pallas import tpu as pltpu
from jax.experimental.pallas import tpu_sc as plsc

VOCAB = 100000
D = 64
B = 1024
CTX = 10

VT = 4096
NV = -(-VOCAB // VT)
VPAD = NV * VT
K = 128

LOG2E = 1.4426950408889634

_GATHER_WIN = 128
_STAT_CHUNK = 8192


def _sc_gather(emb2, x_flat):
    n = x_flat.shape[1]
    mesh = plsc.VectorSubcoreMesh(core_axis_name="c", subcore_axis_name="s")

    @pl.kernel(
        out_type=jax.ShapeDtypeStruct((n, 2 * D), emb2.dtype),
        mesh=mesh,
        scratch_types=[pltpu.VMEM((1, _GATHER_WIN), jnp.int32)],
    )
    def gather_kernel(emb_hbm, i_hbm, o_hbm, tmp_ref):
        def body(i_vmem, o_vmem):
            @pl.loop(0, _GATHER_WIN, step=16)
            def _(c):
                sl = (0, pl.ds(c, 16))
                tmp_ref[sl] = jax.lax.shift_right_logical(i_vmem[sl], 1)

            pltpu.sync_copy(emb_hbm.at[tmp_ref.at[0]], o_vmem)

        pltpu.emit_pipeline(
            body,
            grid=(n // _GATHER_WIN,),
            in_specs=[pl.BlockSpec((1, _GATHER_WIN), index_map=lambda i: (0, i))],
            out_specs=[pl.BlockSpec((_GATHER_WIN, 2 * D), index_map=lambda i: (i, 0))],
            core_axis_name=("c", "s"),
            dimension_semantics=(pltpu.PARALLEL,),
        )(i_hbm, o_hbm)

    return gather_kernel(emb2, x_flat)


def _stats_body(w_ref, o_ref):
    m = jnp.float32(0.0)
    for k in range(VPAD // _STAT_CHUNK):
        c = w_ref[k * _STAT_CHUNK:(k + 1) * _STAT_CHUNK, 0:D].astype(jnp.float32)
        m = jnp.maximum(m, jnp.max(jnp.sum(c * c, axis=1)))
    mb = jnp.max(w_ref[:, D:D + 1].astype(jnp.float32))
    lane = jax.lax.broadcasted_iota(jnp.int32, (1, 128), 1)
    o_ref[...] = jnp.where(lane == 0, jnp.sqrt(m),
                           jnp.where(lane == 1, mb, 0.0))


def _stats_call(w_aug):
    return pl.pallas_call(
        _stats_body,
        out_shape=jax.ShapeDtypeStruct((1, 128), jnp.float32),
    )(w_aug)


def _pool_body(stats_ref, g_ref, x_ref, s_ref, mhat_ref):
    acc = jnp.zeros((B, D), jnp.float32)
    for j in range(CTX):
        left = g_ref[:, j * 2 * D:j * 2 * D + D]
        right = g_ref[:, j * 2 * D + D:(j + 1) * 2 * D]
        odd = (x_ref[:, j:j + 1] & 1) == 1
        acc = acc + jnp.where(odd, right, left)
    lane = jax.lax.broadcasted_iota(jnp.int32, (1, K), 1)
    sa = jnp.pad(acc, ((0, 0), (0, K - D))).astype(jnp.bfloat16)
    s_ref[...] = jnp.where(lane == D, jnp.bfloat16(1.0), sa)
    snorm = jnp.sqrt(jnp.sum(acc * acc, axis=1, keepdims=True))
    mhat_ref[...] = snorm * stats_ref[0, 0] + stats_ref[0, 1]


def _pool_call(stats, g2, x):
    return pl.pallas_call(
        _pool_body,
        out_shape=(
            jax.ShapeDtypeStruct((B, K), jnp.bfloat16),
            jax.ShapeDtypeStruct((B, 1), jnp.float32),
        ),
    )(stats, g2, x)


def _sumexp_body(s_ref, mhat_ref, wt_ref, lse_ref, acc_ref):
    v = pl.program_id(0)

    @pl.when(v == 0)
    def _():
        acc_ref[...] = jnp.zeros((B, 128), jnp.float32)

    t = jax.lax.dot_general(
        s_ref[...], wt_ref[...],
        (((1,), (0,)), ((), ())),
        preferred_element_type=jnp.float32,
    )
    mh2 = mhat_ref[...] * LOG2E
    e2 = jnp.exp2(t * LOG2E - mh2)
    part = e2[:, 0:128]
    for k in range(1, VT // 128):
        part = part + e2[:, k * 128:(k + 1) * 128]
    acc_ref[...] += part

    @pl.when(v == NV - 1)
    def _():
        lse_ref[...] = mhat_ref[...] + jnp.log(
            jnp.sum(acc_ref[...], axis=1, keepdims=True))


def _sumexp_call(s_aug, mhat, wt):
    return pl.pallas_call(
        _sumexp_body,
        grid=(NV,),
        in_specs=[
            pl.BlockSpec((B, K), lambda v: (0, 0)),
            pl.BlockSpec((B, 1), lambda v: (0, 0)),
            pl.BlockSpec((K, VT), lambda v: (0, v)),
        ],
        out_specs=pl.BlockSpec((B, 1), lambda v: (0, 0)),
        out_shape=jax.ShapeDtypeStruct((B, 1), jnp.float32),
        scratch_shapes=[pltpu.VMEM((B, 128), jnp.float32)],
        compiler_params=pltpu.CompilerParams(
            dimension_semantics=("arbitrary",),
        ),
    )(s_aug, mhat, wt)


def _write_body(s_ref, lse_ref, wt_ref, o_ref):
    t = jax.lax.dot_general(
        s_ref[...], wt_ref[...],
        (((1,), (0,)), ((), ())),
        preferred_element_type=jnp.float32,
    )
    o_ref[...] = t - lse_ref[...]


def _write_call(s_aug, lse, wt):
    return pl.pallas_call(
        _write_body,
        grid=(NV,),
        in_specs=[
            pl.BlockSpec((B, K), lambda v: (0, 0)),
            pl.BlockSpec((B, 1), lambda v: (0, 0)),
            pl.BlockSpec((K, VT), lambda v: (0, v)),
        ],
        out_specs=pl.BlockSpec((B, VT), lambda v: (0, v)),
        out_shape=jax.ShapeDtypeStruct((B, VOCAB), jnp.float32),
        compiler_params=pltpu.CompilerParams(
            dimension_semantics=("arbitrary",),
        ),
    )(s_aug, lse, wt)


def kernel(x, emb, W, b):
    x = x.astype(jnp.int32)
    x_flat = x.reshape(1, B * CTX)
    emb2 = emb.reshape(VOCAB // 2, 2 * D)
    g = _sc_gather(emb2, x_flat)
    g2 = g.reshape(B, CTX * 2 * D)

    w_aug = jnp.concatenate([W, b[:, None]], axis=1).astype(jnp.bfloat16)
    w_aug = jnp.pad(w_aug, ((0, VPAD - VOCAB), (0, K - (D + 1))))
    w_aug = w_aug.at[VOCAB:, D].set(jnp.bfloat16(-1e9))
    stats = _stats_call(w_aug)
    wt = w_aug.T

    s_aug, mhat = _pool_call(stats, g2, x)
    lse = _sumexp_call(s_aug, mhat, wt)
    return _write_call(s_aug, lse, wt)
